# baseline (speedup 1.0000x reference)
import functools
import math

import jax
import jax.numpy as jnp
import numpy as np
from jax import lax
from jax.experimental import pallas as pl
from jax.experimental.pallas import tpu as pltpu
from jax.experimental.pallas import tpu_sc as plsc

N = 10000
E = 160000
NUM_ATOM_TYPES = 4
ATOM_EMB = 32
MUL_OUT = 16
NUM_BASIS = 32
HIDDEN = 32
MAX_RADIUS = 2.5
OUT_DIM = 144
TDIM = 128
GDIM = 48
PDIM = 16

NC, NS = 2, 16
NW = NC * NS
EPAD = 163840
ECH = EPAD // 4
CG = 160
CS = 160
SGA = 640
SGB = N - (NS - 1) * SGA
ZR = 40

BE = 2048
KDIM = HIDDEN * ATOM_EMB

_DEGS = (1, 3, 5)
_SH_OFF = (0, 1, 4)


def _layout_maps():
    col_src = []
    perm = np.zeros((OUT_DIM, OUT_DIM), np.float32)
    base = 0
    for l, deg in enumerate(_DEGS):
        for m in range(deg):
            for v in range(MUL_OUT):
                colp = base + m * MUL_OUT + v
                col = base + v * deg + m
                col_src.append(l * MUL_OUT + v)
                perm[colp, col] = 1.0
        base += deg * MUL_OUT
    return np.array(col_src, np.int32), perm


_COL_SRC, _PERM = _layout_maps()


def _build_expanders():
    rh = np.zeros((HIDDEN, KDIM), np.float32)
    rx = np.zeros((ATOM_EMB, KDIM), np.float32)
    for k in range(HIDDEN):
        for u in range(ATOM_EMB):
            rh[k, k * ATOM_EMB + u] = 1.0
            rx[u, k * ATOM_EMB + u] = 1.0
    q = np.zeros((9, 256), np.float32)
    for j in range(9):
        q[j, j * MUL_OUT:(j + 1) * MUL_OUT] = 1.0
    return rh, rx, q


_RH, _RX, _Q256 = _build_expanders()


def _embed_body(z_ref, pos_ref, we_ref, be_ref, t_ref):
    x = jnp.dot(z_ref[...], we_ref[...], preferred_element_type=jnp.float32)
    x = x + be_ref[...]
    pos = pos_ref[...]
    zpad = jnp.zeros((N, TDIM - ATOM_EMB - 3), jnp.float32)
    t_ref[...] = jnp.concatenate([x, pos, zpad], axis=1)


_embed_call = pl.pallas_call(
    _embed_body,
    out_shape=jax.ShapeDtypeStruct((N, TDIM), jnp.float32),
)


@functools.cache
def _make_sc_kernels(ec):
    epw = ec // NW
    eps = ec // NS
    mesh = plsc.VectorSubcoreMesh(core_axis_name="c", subcore_axis_name="s",
                                  num_cores=NC, num_subcores=NS)

    @functools.partial(
        pl.kernel,
        out_type=jax.ShapeDtypeStruct((ec, TDIM), jnp.float32),
        mesh=mesh,
        scratch_types=[pltpu.VMEM((CG,), jnp.int32),
                       pltpu.VMEM((CG,), jnp.int32),
                       pltpu.VMEM((CG, TDIM), jnp.float32),
                       pltpu.VMEM((CG, TDIM), jnp.float32),
                       pltpu.SemaphoreType.DMA,
                       pltpu.SemaphoreType.DMA],
    )
    def gather_kernel(t_hbm, src_hbm, dst_hbm, erow_hbm,
                      isrc_v, idst_v, grow_v, prow_v, sem1, sem2):
        cid = lax.axis_index("c")
        sid = lax.axis_index("s")
        wid = sid * NC + cid

        @pl.loop(0, epw // CG)
        def _(j):
            base = wid * epw + j * CG
            pltpu.sync_copy(src_hbm.at[pl.ds(base, CG)], isrc_v)
            pltpu.sync_copy(dst_hbm.at[pl.ds(base, CG)], idst_v)
            c1 = pltpu.async_copy(t_hbm.at[isrc_v], grow_v, sem1)
            c2 = pltpu.async_copy(t_hbm.at[idst_v], prow_v, sem2)
            c1.wait()
            c2.wait()

            @pl.loop(0, CG)
            def _(i):
                grow_v[i, pl.ds(ATOM_EMB, 16)] = (
                    grow_v[i, pl.ds(ATOM_EMB, 16)]
                    - prow_v[i, pl.ds(ATOM_EMB, 16)])

            pltpu.sync_copy(grow_v, erow_hbm.at[pl.ds(base, CG)])

    @functools.partial(
        pl.kernel,
        out_type=jax.ShapeDtypeStruct((NC, N, 128), jnp.float32),
        mesh=mesh,
        scratch_types=[pltpu.VMEM((CS,), jnp.int32),
                       pltpu.VMEM((CS, 128), jnp.float32),
                       pltpu.VMEM((ZR, 128), jnp.float32),
                       pltpu.VMEM_SHARED((N, 128), jnp.float32),
                       pltpu.SemaphoreType.DMA],
    )
    def scatter_kernel(eoa_hbm, eob_hbm, dst_hbm, out_hbm, idx_v, rows_v,
                       zbuf_v, acc_sh, sem):
        cid = lax.axis_index("c")
        sid = lax.axis_index("s")

        zeros16 = jnp.zeros((16,), jnp.float32)

        @pl.loop(0, ZR)
        def _(i):
            @pl.loop(0, 8)
            def _(k):
                zbuf_v[i, pl.ds(k * 16, 16)] = zeros16

        @pl.when(sid < NS - 1)
        def _():
            @pl.loop(0, SGA // ZR)
            def _(t):
                off = pl.multiple_of(sid * SGA + t * ZR, 8)
                pltpu.sync_copy(zbuf_v, acc_sh.at[pl.ds(off, ZR)])

        @pl.when(sid == NS - 1)
        def _():
            @pl.loop(0, SGB // ZR)
            def _(t):
                pltpu.sync_copy(
                    zbuf_v, acc_sh.at[pl.ds((NS - 1) * SGA + t * ZR, ZR)])

        plsc.subcore_barrier()

        @pl.loop(0, eps // CS)
        def _(j):
            base = sid * eps + j * CS
            pltpu.sync_copy(dst_hbm.at[pl.ds(base, CS)], idx_v)

            @pl.when(cid == 0)
            def _():
                pltpu.sync_copy(eoa_hbm.at[pl.ds(base, CS)], rows_v)

            @pl.when(cid == 1)
            def _():
                pltpu.sync_copy(eob_hbm.at[pl.ds(base, CS)], rows_v)

            pltpu.async_copy(rows_v, acc_sh.at[idx_v], sem, add=True).wait()

        plsc.subcore_barrier()

        @pl.when(sid < NS - 1)
        def _():
            off = pl.multiple_of(sid * SGA, 8)
            pltpu.sync_copy(acc_sh.at[pl.ds(off, SGA)],
                            out_hbm.at[cid, pl.ds(off, SGA)])

        @pl.when(sid == NS - 1)
        def _():
            pltpu.sync_copy(acc_sh.at[pl.ds((NS - 1) * SGA, SGB)],
                            out_hbm.at[cid, pl.ds((NS - 1) * SGA, SGB)])

    return gather_kernel, scatter_kernel


def _dense_body(erow_ref, w1_ref, w2e_ref, rh_ref, rx_ref, q_ref,
                outa_ref, outb_ref, p_scr):
    g = erow_ref[...]
    xs = g[:, 0:ATOM_EMB]
    vec = g[:, ATOM_EMB:ATOM_EMB + 3]
    r2 = jnp.sum(vec * vec, axis=1, keepdims=True)
    r = jnp.sqrt(r2)
    mask = r > 0.0
    r_safe = jnp.where(mask, r, 1.0)
    inv = 1.0 / r_safe
    unit = vec * inv
    ux, uy, uz = unit[:, 0:1], unit[:, 1:2], unit[:, 2:3]

    c15 = math.sqrt(15.0)
    c5 = math.sqrt(5.0)
    c3 = math.sqrt(3.0)
    sh9 = jnp.concatenate([
        jnp.ones((BE, 1), jnp.float32),
        c3 * ux, c3 * uy, c3 * uz,
        c15 * ux * uz,
        c15 * ux * uy,
        c5 * (uy * uy - 0.5 * (ux * ux + uz * uz)),
        c15 * uy * uz,
        (c15 / 2.0) * (uz * uz - ux * ux),
    ], axis=1)
    sh_exp = jnp.dot(sh9.astype(jnp.bfloat16), q_ref[...],
                     preferred_element_type=jnp.float32)

    nvec = jnp.arange(1, NUM_BASIS + 1, dtype=jnp.int32)[None, :].astype(
        jnp.float32)
    rc = jnp.minimum(r, MAX_RADIUS)
    tn = (0.5 / MAX_RADIUS) * rc * nvec
    u = tn - jnp.floor(tn + 0.5)
    w = (2.0 * math.pi) * u
    w2 = w * w
    poly = -1.0 / 39916800.0 + w2 * (1.0 / 6227020800.0
                                     - w2 * (1.0 / 1307674368000.0))
    s = w * (1.0 + w2 * (-1.0 / 6.0 + w2 * (1.0 / 120.0 + w2 * (
        -1.0 / 5040.0 + w2 * (1.0 / 362880.0 + w2 * poly)))))
    keep = mask & (r < MAX_RADIUS)
    basis = jnp.where(keep, math.sqrt(2.0 / MAX_RADIUS) * s * inv, 0.0)

    hpre = jnp.dot(basis.astype(jnp.bfloat16), w1_ref[...],
                   preferred_element_type=jnp.float32)
    h = hpre * jax.nn.sigmoid(hpre)

    hrep = jnp.dot(h.astype(jnp.bfloat16), rh_ref[...],
                   preferred_element_type=jnp.float32)
    xrep = jnp.dot(xs.astype(jnp.bfloat16), rx_ref[...],
                   preferred_element_type=jnp.float32)
    p_scr[...] = hrep.astype(jnp.bfloat16) * xrep.astype(jnp.bfloat16)
    o = jnp.dot(p_scr[...], w2e_ref[...], preferred_element_type=jnp.float32)
    eo = o * sh_exp
    outa_ref[...] = eo[:, 0:128]
    outb_ref[...] = eo[:, 128:256]


@functools.cache
def _make_dense_call(ec):
    return pl.pallas_call(
        _dense_body,
        grid=(ec // BE,),
        in_specs=[pl.BlockSpec((BE, TDIM), lambda i: (i, 0)),
                  pl.BlockSpec((NUM_BASIS, HIDDEN), lambda i: (0, 0)),
                  pl.BlockSpec((KDIM, 256), lambda i: (0, 0)),
                  pl.BlockSpec((HIDDEN, KDIM), lambda i: (0, 0)),
                  pl.BlockSpec((ATOM_EMB, KDIM), lambda i: (0, 0)),
                  pl.BlockSpec((9, 256), lambda i: (0, 0))],
        out_specs=[pl.BlockSpec((BE, 128), lambda i: (i, 0)),
                   pl.BlockSpec((BE, 128), lambda i: (i, 0))],
        out_shape=[jax.ShapeDtypeStruct((ec, 128), jnp.float32),
                   jax.ShapeDtypeStruct((ec, 128), jnp.float32)],
        scratch_shapes=[pltpu.VMEM((BE, KDIM), jnp.bfloat16)],
        compiler_params=pltpu.CompilerParams(
            dimension_semantics=("parallel",)),
    )


def _combine_body(p1_ref, p2_ref, p3_ref, p4_ref, perm_ref, out_ref):
    refs = (p1_ref, p2_ref, p3_ref, p4_ref)
    pa = sum(p[0] for p in refs[1:]) + refs[0][0]
    pb = sum(p[1] for p in refs[1:]) + refs[0][1]
    acc = jnp.concatenate([pa, pb[:, 0:OUT_DIM - 128]], axis=1)
    out_ref[...] = jnp.dot(acc, perm_ref[...],
                           preferred_element_type=jnp.float32)


_combine_call = pl.pallas_call(
    _combine_body,
    out_shape=jax.ShapeDtypeStruct((N, OUT_DIM), jnp.float32),
)


def kernel(z, pos, edge_index, W_embed, b_embed, W1, W2):
    src = edge_index[0].astype(jnp.int32)
    dst = edge_index[1].astype(jnp.int32)

    w1b = (W1 / math.sqrt(NUM_BASIS)).astype(jnp.bfloat16)
    w2r = jnp.transpose(
        W2.reshape(HIDDEN, 3, ATOM_EMB, MUL_OUT), (0, 2, 1, 3)
    ).reshape(KDIM, 3 * MUL_OUT) / (math.sqrt(HIDDEN) * math.sqrt(ATOM_EMB))
    w2e = jnp.concatenate(
        [jnp.take(w2r, jnp.asarray(_COL_SRC), axis=1),
         jnp.zeros((KDIM, 256 - OUT_DIM), jnp.float32)],
        axis=1).astype(jnp.bfloat16)
    rh = jnp.asarray(_RH).astype(jnp.bfloat16)
    rx = jnp.asarray(_RX).astype(jnp.bfloat16)
    q256 = jnp.asarray(_Q256).astype(jnp.bfloat16)
    perm = jnp.asarray(_PERM)

    t_tab = _embed_call(z, pos, W_embed, b_embed.reshape(1, ATOM_EMB))

    pad = jnp.zeros((EPAD - E,), jnp.int32)
    srcp = jnp.concatenate([src, pad])
    dstp = jnp.concatenate([dst, pad])
    gather_kernel, scatter_kernel = _make_sc_kernels(ECH)
    dense_call = _make_dense_call(ECH)
    parts = []
    for i in range(4):
        s_c = lax.slice_in_dim(srcp, i * ECH, (i + 1) * ECH)
        d_c = lax.slice_in_dim(dstp, i * ECH, (i + 1) * ECH)
        erow = gather_kernel(t_tab, s_c, d_c)
        eoa, eob = dense_call(erow, w1b, w2e, rh, rx, q256)
        parts.append(scatter_kernel(eoa, eob, d_c))
    return _combine_call(*parts, perm)


# speedup vs baseline: 1.2860x; 1.2860x over previous
import functools
import math

import jax
import jax.numpy as jnp
import numpy as np
from jax import lax
from jax.experimental import pallas as pl
from jax.experimental.pallas import tpu as pltpu
from jax.experimental.pallas import tpu_sc as plsc

N = 10000
E = 160000
NUM_ATOM_TYPES = 4
ATOM_EMB = 32
MUL_OUT = 16
NUM_BASIS = 32
HIDDEN = 32
MAX_RADIUS = 2.5
OUT_DIM = 144
TDIM = 128
GDIM = 48
PDIM = 16

NC, NS = 2, 16
NW = NC * NS
E1 = 96000
CG = 200
CS = 200
SGA = 640
SGB = N - (NS - 1) * SGA
ZR = 40

BE = 2000
KDIM = HIDDEN * ATOM_EMB

_DEGS = (1, 3, 5)
_SH_OFF = (0, 1, 4)


def _layout_maps():
    col_src = []
    perm = np.zeros((OUT_DIM, OUT_DIM), np.float32)
    base = 0
    for l, deg in enumerate(_DEGS):
        for m in range(deg):
            for v in range(MUL_OUT):
                colp = base + m * MUL_OUT + v
                col = base + v * deg + m
                col_src.append(l * MUL_OUT + v)
                perm[colp, col] = 1.0
        base += deg * MUL_OUT
    return np.array(col_src, np.int32), perm


_COL_SRC, _PERM = _layout_maps()


def _build_expanders():
    rh = np.zeros((HIDDEN, KDIM), np.float32)
    rx = np.zeros((ATOM_EMB, KDIM), np.float32)
    for k in range(HIDDEN):
        for u in range(ATOM_EMB):
            rh[k, k * ATOM_EMB + u] = 1.0
            rx[u, k * ATOM_EMB + u] = 1.0
    q = np.zeros((9, 256), np.float32)
    for j in range(9):
        q[j, j * MUL_OUT:(j + 1) * MUL_OUT] = 1.0
    return rh, rx, q


_RH, _RX, _Q256 = _build_expanders()


def _embed_body(z_ref, pos_ref, we_ref, be_ref, t_ref):
    x = jnp.dot(z_ref[...], we_ref[...], preferred_element_type=jnp.float32)
    x = x + be_ref[...]
    pos = pos_ref[...]
    zpad = jnp.zeros((N, TDIM - ATOM_EMB - 3), jnp.float32)
    t_ref[...] = jnp.concatenate([x, pos, zpad], axis=1)


_embed_call = pl.pallas_call(
    _embed_body,
    out_shape=jax.ShapeDtypeStruct((N, TDIM), jnp.float32),
)


@functools.cache
def _make_sc_kernels(ec):
    epw = ec // NW
    eps = ec // NS
    mesh = plsc.VectorSubcoreMesh(core_axis_name="c", subcore_axis_name="s",
                                  num_cores=NC, num_subcores=NS)

    @functools.partial(
        pl.kernel,
        out_type=jax.ShapeDtypeStruct((ec, TDIM), jnp.float32),
        mesh=mesh,
        scratch_types=[pltpu.VMEM((CG,), jnp.int32),
                       pltpu.VMEM((CG,), jnp.int32),
                       pltpu.VMEM((CG, TDIM), jnp.float32),
                       pltpu.VMEM((CG, TDIM), jnp.float32),
                       pltpu.SemaphoreType.DMA,
                       pltpu.SemaphoreType.DMA],
    )
    def gather_kernel(t_hbm, src_hbm, dst_hbm, erow_hbm,
                      isrc_v, idst_v, grow_v, prow_v, sem1, sem2):
        cid = lax.axis_index("c")
        sid = lax.axis_index("s")
        wid = sid * NC + cid

        @pl.loop(0, epw // CG)
        def _(j):
            base = wid * epw + j * CG
            pltpu.sync_copy(src_hbm.at[pl.ds(base, CG)], isrc_v)
            pltpu.sync_copy(dst_hbm.at[pl.ds(base, CG)], idst_v)
            c1 = pltpu.async_copy(t_hbm.at[isrc_v], grow_v, sem1)
            c2 = pltpu.async_copy(t_hbm.at[idst_v], prow_v, sem2)
            c1.wait()
            c2.wait()

            @pl.loop(0, CG)
            def _(i):
                grow_v[i, pl.ds(ATOM_EMB, 16)] = (
                    grow_v[i, pl.ds(ATOM_EMB, 16)]
                    - prow_v[i, pl.ds(ATOM_EMB, 16)])

            pltpu.sync_copy(grow_v, erow_hbm.at[pl.ds(base, CG)])

    @functools.partial(
        pl.kernel,
        out_type=jax.ShapeDtypeStruct((NC, N, 128), jnp.float32),
        mesh=mesh,
        scratch_types=[pltpu.VMEM((CS,), jnp.int32),
                       pltpu.VMEM((CS, 128), jnp.float32),
                       pltpu.VMEM((ZR, 128), jnp.float32),
                       pltpu.VMEM_SHARED((N, 128), jnp.float32),
                       pltpu.SemaphoreType.DMA],
    )
    def scatter_kernel(eoa_hbm, eob_hbm, dst_hbm, out_hbm, idx_v, rows_v,
                       zbuf_v, acc_sh, sem):
        cid = lax.axis_index("c")
        sid = lax.axis_index("s")

        zeros16 = jnp.zeros((16,), jnp.float32)

        @pl.loop(0, ZR)
        def _(i):
            @pl.loop(0, 128 // 16)
            def _(j):
                zbuf_v[i, pl.ds(j * 16, 16)] = zeros16

        @pl.when(sid < NS - 1)
        def _():
            @pl.loop(0, SGA // ZR)
            def _(t):
                off = pl.multiple_of(sid * SGA + t * ZR, 8)
                pltpu.sync_copy(zbuf_v, acc_sh.at[pl.ds(off, ZR)])

        @pl.when(sid == NS - 1)
        def _():
            @pl.loop(0, SGB // ZR)
            def _(t):
                pltpu.sync_copy(
                    zbuf_v, acc_sh.at[pl.ds((NS - 1) * SGA + t * ZR, ZR)])

        plsc.subcore_barrier()

        @pl.loop(0, eps // CS)
        def _(j):
            base = sid * eps + j * CS
            pltpu.sync_copy(dst_hbm.at[pl.ds(base, CS)], idx_v)

            @pl.when(cid == 0)
            def _():
                pltpu.sync_copy(eoa_hbm.at[pl.ds(base, CS)], rows_v)

            @pl.when(cid == 1)
            def _():
                pltpu.sync_copy(eob_hbm.at[pl.ds(base, CS)], rows_v)

            pltpu.async_copy(rows_v, acc_sh.at[idx_v], sem, add=True).wait()

        plsc.subcore_barrier()

        @pl.when(sid < NS - 1)
        def _():
            off = pl.multiple_of(sid * SGA, 8)
            pltpu.sync_copy(acc_sh.at[pl.ds(off, SGA)],
                            out_hbm.at[cid, pl.ds(off, SGA)])

        @pl.when(sid == NS - 1)
        def _():
            pltpu.sync_copy(acc_sh.at[pl.ds((NS - 1) * SGA, SGB)],
                            out_hbm.at[cid, pl.ds((NS - 1) * SGA, SGB)])

    return gather_kernel, scatter_kernel


def _dense_body(erow_ref, w1_ref, w2e_ref, rh_ref, rx_ref, q_ref,
                outa_ref, outb_ref, p_scr):
    g = erow_ref[...]
    xs = g[:, 0:ATOM_EMB]
    vec = g[:, ATOM_EMB:ATOM_EMB + 3]
    r2 = jnp.sum(vec * vec, axis=1, keepdims=True)
    r = jnp.sqrt(r2)
    mask = r > 0.0
    r_safe = jnp.where(mask, r, 1.0)
    inv = 1.0 / r_safe
    unit = vec * inv
    ux, uy, uz = unit[:, 0:1], unit[:, 1:2], unit[:, 2:3]

    c15 = math.sqrt(15.0)
    c5 = math.sqrt(5.0)
    c3 = math.sqrt(3.0)
    sh9 = jnp.concatenate([
        jnp.ones((BE, 1), jnp.float32),
        c3 * ux, c3 * uy, c3 * uz,
        c15 * ux * uz,
        c15 * ux * uy,
        c5 * (uy * uy - 0.5 * (ux * ux + uz * uz)),
        c15 * uy * uz,
        (c15 / 2.0) * (uz * uz - ux * ux),
    ], axis=1)
    sh_exp = jnp.dot(sh9.astype(jnp.bfloat16), q_ref[...],
                     preferred_element_type=jnp.float32)

    nvec = jnp.arange(1, NUM_BASIS + 1, dtype=jnp.int32)[None, :].astype(
        jnp.float32)
    rc = jnp.minimum(r, MAX_RADIUS)
    tn = (0.5 / MAX_RADIUS) * rc * nvec
    u = tn - jnp.floor(tn + 0.5)
    w = (2.0 * math.pi) * u
    w2 = w * w
    poly = -1.0 / 39916800.0 + w2 * (1.0 / 6227020800.0
                                     - w2 * (1.0 / 1307674368000.0))
    s = w * (1.0 + w2 * (-1.0 / 6.0 + w2 * (1.0 / 120.0 + w2 * (
        -1.0 / 5040.0 + w2 * (1.0 / 362880.0 + w2 * poly)))))
    keep = mask & (r < MAX_RADIUS)
    basis = jnp.where(keep, math.sqrt(2.0 / MAX_RADIUS) * s * inv, 0.0)

    hpre = jnp.dot(basis.astype(jnp.bfloat16), w1_ref[...],
                   preferred_element_type=jnp.float32)
    h = hpre * jax.nn.sigmoid(hpre)

    hrep = jnp.dot(h.astype(jnp.bfloat16), rh_ref[...],
                   preferred_element_type=jnp.float32)
    xrep = jnp.dot(xs.astype(jnp.bfloat16), rx_ref[...],
                   preferred_element_type=jnp.float32)
    p_scr[...] = hrep.astype(jnp.bfloat16) * xrep.astype(jnp.bfloat16)
    o = jnp.dot(p_scr[...], w2e_ref[...], preferred_element_type=jnp.float32)
    eo = o * sh_exp
    outa_ref[...] = eo[:, 0:128]
    outb_ref[...] = eo[:, 128:256]


@functools.cache
def _make_dense_call(ec):
    return pl.pallas_call(
        _dense_body,
        grid=(ec // BE,),
        in_specs=[pl.BlockSpec((BE, TDIM), lambda i: (i, 0)),
                  pl.BlockSpec((NUM_BASIS, HIDDEN), lambda i: (0, 0)),
                  pl.BlockSpec((KDIM, 256), lambda i: (0, 0)),
                  pl.BlockSpec((HIDDEN, KDIM), lambda i: (0, 0)),
                  pl.BlockSpec((ATOM_EMB, KDIM), lambda i: (0, 0)),
                  pl.BlockSpec((9, 256), lambda i: (0, 0))],
        out_specs=[pl.BlockSpec((BE, 128), lambda i: (i, 0)),
                   pl.BlockSpec((BE, 128), lambda i: (i, 0))],
        out_shape=[jax.ShapeDtypeStruct((ec, 128), jnp.float32),
                   jax.ShapeDtypeStruct((ec, 128), jnp.float32)],
        scratch_shapes=[pltpu.VMEM((BE, KDIM), jnp.bfloat16)],
        compiler_params=pltpu.CompilerParams(
            dimension_semantics=("parallel",)),
    )


def _combine_body(p1_ref, p2_ref, perm_ref, out_ref):
    pa = p1_ref[0] + p2_ref[0]
    pb = p1_ref[1] + p2_ref[1]
    acc = jnp.concatenate([pa, pb[:, 0:OUT_DIM - 128]], axis=1)
    out_ref[...] = jnp.dot(acc, perm_ref[...],
                           preferred_element_type=jnp.float32)


_combine_call = pl.pallas_call(
    _combine_body,
    out_shape=jax.ShapeDtypeStruct((N, OUT_DIM), jnp.float32),
)


def kernel(z, pos, edge_index, W_embed, b_embed, W1, W2):
    src = edge_index[0].astype(jnp.int32)
    dst = edge_index[1].astype(jnp.int32)

    w1b = (W1 / math.sqrt(NUM_BASIS)).astype(jnp.bfloat16)
    w2r = jnp.transpose(
        W2.reshape(HIDDEN, 3, ATOM_EMB, MUL_OUT), (0, 2, 1, 3)
    ).reshape(KDIM, 3 * MUL_OUT) / (math.sqrt(HIDDEN) * math.sqrt(ATOM_EMB))
    w2e = jnp.concatenate(
        [jnp.take(w2r, jnp.asarray(_COL_SRC), axis=1),
         jnp.zeros((KDIM, 256 - OUT_DIM), jnp.float32)],
        axis=1).astype(jnp.bfloat16)
    rh = jnp.asarray(_RH).astype(jnp.bfloat16)
    rx = jnp.asarray(_RX).astype(jnp.bfloat16)
    q256 = jnp.asarray(_Q256).astype(jnp.bfloat16)
    perm = jnp.asarray(_PERM)

    t_tab = _embed_call(z, pos, W_embed, b_embed.reshape(1, ATOM_EMB))

    parts = []
    for lo, ec in ((0, E1), (E1, E - E1)):
        gather_kernel, scatter_kernel = _make_sc_kernels(ec)
        s_c = lax.slice_in_dim(src, lo, lo + ec)
        d_c = lax.slice_in_dim(dst, lo, lo + ec)
        erow = gather_kernel(t_tab, s_c, d_c)
        eoa, eob = _make_dense_call(ec)(erow, w1b, w2e, rh, rx, q256)
        parts.append(scatter_kernel(eoa, eob, d_c))
    return _combine_call(parts[0], parts[1], perm)


# speedup vs baseline: 1.2924x; 1.0049x over previous
import functools
import math

import jax
import jax.numpy as jnp
import numpy as np
from jax import lax
from jax.experimental import pallas as pl
from jax.experimental.pallas import tpu as pltpu
from jax.experimental.pallas import tpu_sc as plsc

N = 10000
E = 160000
NUM_ATOM_TYPES = 4
ATOM_EMB = 32
MUL_OUT = 16
NUM_BASIS = 32
HIDDEN = 32
MAX_RADIUS = 2.5
OUT_DIM = 144
TDIM = 128
GDIM = 48
PDIM = 16

NC, NS = 2, 16
NW = NC * NS
E1 = 96000
CG = 200
CS = 200
SGA = 640
SGB = N - (NS - 1) * SGA
ZR = 40

BE = 4000
KDIM = HIDDEN * ATOM_EMB

_DEGS = (1, 3, 5)
_SH_OFF = (0, 1, 4)


def _layout_maps():
    col_src = []
    perm = np.zeros((OUT_DIM, OUT_DIM), np.float32)
    base = 0
    for l, deg in enumerate(_DEGS):
        for m in range(deg):
            for v in range(MUL_OUT):
                colp = base + m * MUL_OUT + v
                col = base + v * deg + m
                col_src.append(l * MUL_OUT + v)
                perm[colp, col] = 1.0
        base += deg * MUL_OUT
    return np.array(col_src, np.int32), perm


_COL_SRC, _PERM = _layout_maps()


def _build_expanders():
    rh = np.zeros((HIDDEN, KDIM), np.float32)
    rx = np.zeros((ATOM_EMB, KDIM), np.float32)
    for k in range(HIDDEN):
        for u in range(ATOM_EMB):
            rh[k, k * ATOM_EMB + u] = 1.0
            rx[u, k * ATOM_EMB + u] = 1.0
    q = np.zeros((9, 256), np.float32)
    for j in range(9):
        q[j, j * MUL_OUT:(j + 1) * MUL_OUT] = 1.0
    return rh, rx, q


_RH, _RX, _Q256 = _build_expanders()


def _embed_body(z_ref, pos_ref, we_ref, be_ref, t_ref):
    x = jnp.dot(z_ref[...], we_ref[...], preferred_element_type=jnp.float32)
    x = x + be_ref[...]
    pos = pos_ref[...]
    zpad = jnp.zeros((N, TDIM - ATOM_EMB - 3), jnp.float32)
    t_ref[...] = jnp.concatenate([x, pos, zpad], axis=1)


_embed_call = pl.pallas_call(
    _embed_body,
    out_shape=jax.ShapeDtypeStruct((N, TDIM), jnp.float32),
)


@functools.cache
def _make_sc_kernels(ec):
    epw = ec // NW
    eps = ec // NS
    mesh = plsc.VectorSubcoreMesh(core_axis_name="c", subcore_axis_name="s",
                                  num_cores=NC, num_subcores=NS)

    @functools.partial(
        pl.kernel,
        out_type=jax.ShapeDtypeStruct((ec, TDIM), jnp.float32),
        mesh=mesh,
        scratch_types=[pltpu.VMEM((CG,), jnp.int32),
                       pltpu.VMEM((CG,), jnp.int32),
                       pltpu.VMEM((CG, TDIM), jnp.float32),
                       pltpu.VMEM((CG, TDIM), jnp.float32),
                       pltpu.SemaphoreType.DMA,
                       pltpu.SemaphoreType.DMA],
    )
    def gather_kernel(t_hbm, src_hbm, dst_hbm, erow_hbm,
                      isrc_v, idst_v, grow_v, prow_v, sem1, sem2):
        cid = lax.axis_index("c")
        sid = lax.axis_index("s")
        wid = sid * NC + cid

        @pl.loop(0, epw // CG)
        def _(j):
            base = wid * epw + j * CG
            pltpu.sync_copy(src_hbm.at[pl.ds(base, CG)], isrc_v)
            pltpu.sync_copy(dst_hbm.at[pl.ds(base, CG)], idst_v)
            c1 = pltpu.async_copy(t_hbm.at[isrc_v], grow_v, sem1)
            c2 = pltpu.async_copy(t_hbm.at[idst_v], prow_v, sem2)
            c1.wait()
            c2.wait()

            @pl.loop(0, CG)
            def _(i):
                grow_v[i, pl.ds(ATOM_EMB, 16)] = (
                    grow_v[i, pl.ds(ATOM_EMB, 16)]
                    - prow_v[i, pl.ds(ATOM_EMB, 16)])

            pltpu.sync_copy(grow_v, erow_hbm.at[pl.ds(base, CG)])

    @functools.partial(
        pl.kernel,
        out_type=jax.ShapeDtypeStruct((NC, N, 128), jnp.float32),
        mesh=mesh,
        scratch_types=[pltpu.VMEM((CS,), jnp.int32),
                       pltpu.VMEM((CS, 128), jnp.float32),
                       pltpu.VMEM((ZR, 128), jnp.float32),
                       pltpu.VMEM_SHARED((N, 128), jnp.float32),
                       pltpu.SemaphoreType.DMA],
    )
    def scatter_kernel(eoa_hbm, eob_hbm, dst_hbm, out_hbm, idx_v, rows_v,
                       zbuf_v, acc_sh, sem):
        cid = lax.axis_index("c")
        sid = lax.axis_index("s")

        zeros16 = jnp.zeros((16,), jnp.float32)

        @pl.loop(0, ZR)
        def _(i):
            @pl.loop(0, 128 // 16)
            def _(j):
                zbuf_v[i, pl.ds(j * 16, 16)] = zeros16

        @pl.when(sid < NS - 1)
        def _():
            @pl.loop(0, SGA // ZR)
            def _(t):
                off = pl.multiple_of(sid * SGA + t * ZR, 8)
                pltpu.sync_copy(zbuf_v, acc_sh.at[pl.ds(off, ZR)])

        @pl.when(sid == NS - 1)
        def _():
            @pl.loop(0, SGB // ZR)
            def _(t):
                pltpu.sync_copy(
                    zbuf_v, acc_sh.at[pl.ds((NS - 1) * SGA + t * ZR, ZR)])

        plsc.subcore_barrier()

        @pl.loop(0, eps // CS)
        def _(j):
            base = sid * eps + j * CS
            pltpu.sync_copy(dst_hbm.at[pl.ds(base, CS)], idx_v)

            @pl.when(cid == 0)
            def _():
                pltpu.sync_copy(eoa_hbm.at[pl.ds(base, CS)], rows_v)

            @pl.when(cid == 1)
            def _():
                pltpu.sync_copy(eob_hbm.at[pl.ds(base, CS)], rows_v)

            pltpu.async_copy(rows_v, acc_sh.at[idx_v], sem, add=True).wait()

        plsc.subcore_barrier()

        @pl.when(sid < NS - 1)
        def _():
            off = pl.multiple_of(sid * SGA, 8)
            pltpu.sync_copy(acc_sh.at[pl.ds(off, SGA)],
                            out_hbm.at[cid, pl.ds(off, SGA)])

        @pl.when(sid == NS - 1)
        def _():
            pltpu.sync_copy(acc_sh.at[pl.ds((NS - 1) * SGA, SGB)],
                            out_hbm.at[cid, pl.ds((NS - 1) * SGA, SGB)])

    return gather_kernel, scatter_kernel


def _dense_body(erow_ref, w1_ref, w2e_ref, rh_ref, rx_ref, q_ref,
                outa_ref, outb_ref, p_scr):
    g = erow_ref[...]
    xs = g[:, 0:ATOM_EMB]
    vec = g[:, ATOM_EMB:ATOM_EMB + 3]
    r2 = jnp.sum(vec * vec, axis=1, keepdims=True)
    r = jnp.sqrt(r2)
    mask = r > 0.0
    r_safe = jnp.where(mask, r, 1.0)
    inv = 1.0 / r_safe
    unit = vec * inv
    ux, uy, uz = unit[:, 0:1], unit[:, 1:2], unit[:, 2:3]

    c15 = math.sqrt(15.0)
    c5 = math.sqrt(5.0)
    c3 = math.sqrt(3.0)
    sh9 = jnp.concatenate([
        jnp.ones((BE, 1), jnp.float32),
        c3 * ux, c3 * uy, c3 * uz,
        c15 * ux * uz,
        c15 * ux * uy,
        c5 * (uy * uy - 0.5 * (ux * ux + uz * uz)),
        c15 * uy * uz,
        (c15 / 2.0) * (uz * uz - ux * ux),
    ], axis=1)
    sh_exp = jnp.dot(sh9.astype(jnp.bfloat16), q_ref[...],
                     preferred_element_type=jnp.float32)

    nvec = jnp.arange(1, NUM_BASIS + 1, dtype=jnp.int32)[None, :].astype(
        jnp.float32)
    rc = jnp.minimum(r, MAX_RADIUS)
    tn = (0.5 / MAX_RADIUS) * rc * nvec
    u = tn - jnp.floor(tn + 0.5)
    w = (2.0 * math.pi) * u
    w2 = w * w
    poly = -1.0 / 39916800.0 + w2 * (1.0 / 6227020800.0
                                     - w2 * (1.0 / 1307674368000.0))
    s = w * (1.0 + w2 * (-1.0 / 6.0 + w2 * (1.0 / 120.0 + w2 * (
        -1.0 / 5040.0 + w2 * (1.0 / 362880.0 + w2 * poly)))))
    keep = mask & (r < MAX_RADIUS)
    basis = jnp.where(keep, math.sqrt(2.0 / MAX_RADIUS) * s * inv, 0.0)

    hpre = jnp.dot(basis.astype(jnp.bfloat16), w1_ref[...],
                   preferred_element_type=jnp.float32)
    h = hpre * jax.nn.sigmoid(hpre)

    hrep = jnp.dot(h.astype(jnp.bfloat16), rh_ref[...],
                   preferred_element_type=jnp.float32)
    xrep = jnp.dot(xs.astype(jnp.bfloat16), rx_ref[...],
                   preferred_element_type=jnp.float32)
    p_scr[...] = hrep.astype(jnp.bfloat16) * xrep.astype(jnp.bfloat16)
    o = jnp.dot(p_scr[...], w2e_ref[...], preferred_element_type=jnp.float32)
    eo = o * sh_exp
    outa_ref[...] = eo[:, 0:128]
    outb_ref[...] = eo[:, 128:256]


@functools.cache
def _make_dense_call(ec):
    return pl.pallas_call(
        _dense_body,
        grid=(ec // BE,),
        in_specs=[pl.BlockSpec((BE, TDIM), lambda i: (i, 0)),
                  pl.BlockSpec((NUM_BASIS, HIDDEN), lambda i: (0, 0)),
                  pl.BlockSpec((KDIM, 256), lambda i: (0, 0)),
                  pl.BlockSpec((HIDDEN, KDIM), lambda i: (0, 0)),
                  pl.BlockSpec((ATOM_EMB, KDIM), lambda i: (0, 0)),
                  pl.BlockSpec((9, 256), lambda i: (0, 0))],
        out_specs=[pl.BlockSpec((BE, 128), lambda i: (i, 0)),
                   pl.BlockSpec((BE, 128), lambda i: (i, 0))],
        out_shape=[jax.ShapeDtypeStruct((ec, 128), jnp.float32),
                   jax.ShapeDtypeStruct((ec, 128), jnp.float32)],
        scratch_shapes=[pltpu.VMEM((BE, KDIM), jnp.bfloat16)],
        compiler_params=pltpu.CompilerParams(
            dimension_semantics=("parallel",)),
    )


def _combine_body(p1_ref, p2_ref, perm_ref, out_ref):
    pa = p1_ref[0] + p2_ref[0]
    pb = p1_ref[1] + p2_ref[1]
    acc = jnp.concatenate([pa, pb[:, 0:OUT_DIM - 128]], axis=1)
    out_ref[...] = jnp.dot(acc, perm_ref[...],
                           preferred_element_type=jnp.float32)


_combine_call = pl.pallas_call(
    _combine_body,
    out_shape=jax.ShapeDtypeStruct((N, OUT_DIM), jnp.float32),
)


def kernel(z, pos, edge_index, W_embed, b_embed, W1, W2):
    src = edge_index[0].astype(jnp.int32)
    dst = edge_index[1].astype(jnp.int32)

    w1b = (W1 / math.sqrt(NUM_BASIS)).astype(jnp.bfloat16)
    w2r = jnp.transpose(
        W2.reshape(HIDDEN, 3, ATOM_EMB, MUL_OUT), (0, 2, 1, 3)
    ).reshape(KDIM, 3 * MUL_OUT) / (math.sqrt(HIDDEN) * math.sqrt(ATOM_EMB))
    w2e = jnp.concatenate(
        [jnp.take(w2r, jnp.asarray(_COL_SRC), axis=1),
         jnp.zeros((KDIM, 256 - OUT_DIM), jnp.float32)],
        axis=1).astype(jnp.bfloat16)
    rh = jnp.asarray(_RH).astype(jnp.bfloat16)
    rx = jnp.asarray(_RX).astype(jnp.bfloat16)
    q256 = jnp.asarray(_Q256).astype(jnp.bfloat16)
    perm = jnp.asarray(_PERM)

    t_tab = _embed_call(z, pos, W_embed, b_embed.reshape(1, ATOM_EMB))

    parts = []
    for lo, ec in ((0, E1), (E1, E - E1)):
        gather_kernel, scatter_kernel = _make_sc_kernels(ec)
        s_c = lax.slice_in_dim(src, lo, lo + ec)
        d_c = lax.slice_in_dim(dst, lo, lo + ec)
        erow = gather_kernel(t_tab, s_c, d_c)
        eoa, eob = _make_dense_call(ec)(erow, w1b, w2e, rh, rx, q256)
        parts.append(scatter_kernel(eoa, eob, d_c))
    return _combine_call(parts[0], parts[1], perm)
